# baseline (speedup 1.0000x reference)
import functools

import jax
import jax.numpy as jnp
from jax import lax
from jax.experimental import pallas as pl
from jax.experimental.pallas import tpu as pltpu
from jax.experimental.pallas import tpu_sc as plsc

NC = 2
NS = 16
NW = NC * NS
LANES = 16

K = 80
GCH = 25
NEG_SLOPE = 0.2


def _sc_edge_body(npad, nchunk, d,
                  src_hbm, dst_hbm, as_hbm, ad_hbm, hbf_hbm,
                  outp_hbm, denp_hbm,
                  src_v, dst_v, as_v, ad_v, ex_v, rowsbf_v, rowsf_v,
                  outacc, denacc, gsem0, gsem1, ssem):
    c = lax.axis_index("c")
    s = lax.axis_index("s")
    tid = c * NS + s
    span = npad // NS
    nvec = 128 // LANES
    gsem = (gsem0, gsem1)

    zv = jnp.zeros((LANES,), jnp.float32)

    def _zrows(r, _):
        for v in range(nvec):
            rowsf_v[r, pl.ds(v * LANES, LANES)] = zv
        return 0
    lax.fori_loop(0, K, _zrows, 0)

    def _zex(j, _):
        ex_v[0, pl.ds(j * LANES, LANES)] = zv
        return 0
    lax.fori_loop(0, K // LANES, _zex, 0)

    def _zacc(i, _):
        pltpu.sync_copy(rowsf_v, outacc.at[pl.ds(s * span + i * K, K)])
        return 0
    lax.fori_loop(0, span // K, _zacc, 0)

    def _zden(i, _):
        pltpu.sync_copy(ex_v.at[0], denacc.at[pl.ds(s * span + i * K, K)])
        return 0
    lax.fori_loop(0, span // K, _zden, 0)

    pltpu.sync_copy(as_hbm, as_v)
    pltpu.sync_copy(ad_hbm, ad_v)

    plsc.subcore_barrier()


    def _gissue(l, b):
        pltpu.async_copy(hbf_hbm.at[src_v.at[pl.ds(l * K, K)]],
                         rowsbf_v.at[b], gsem[b])

    def _gwait(b):
        pltpu.make_async_copy(hbf_hbm.at[src_v.at[pl.ds(0, K)]],
                              rowsbf_v.at[b], gsem[b]).wait()

    def _sissue(l, b):
        pltpu.async_copy(rowsf_v, outacc.at[dst_v.at[l]], ssem,
                         add=True)
        pltpu.async_copy(ex_v.at[b], denacc.at[dst_v.at[l]], ssem,
                         add=True)

    def _sdrain(b):
        pltpu.make_async_copy(rowsf_v, outacc.at[dst_v.at[0]],
                              ssem).wait()
        pltpu.make_async_copy(ex_v.at[b], denacc.at[dst_v.at[0]],
                              ssem).wait()

    def _ex_compute(l, b):
        for j in range(K // LANES):
            s16 = src_v[pl.ds(l * K + j * LANES, LANES)]
            d16 = dst_v[l, pl.ds(j * LANES, LANES)]
            av = plsc.load_gather(as_v, [s16])
            dv = plsc.load_gather(ad_v, [d16])
            z = av + dv
            e = jnp.where(z > 0, z, NEG_SLOPE * z)
            ex_v[b, pl.ds(j * LANES, LANES)] = jnp.exp(e)

    def _scale(b):
        def body(q, _):
            exg = ex_v[b, pl.ds(q * LANES, LANES)]
            for t in range(LANES):
                w = exg[t]
                r = q * LANES + t
                for v in range(nvec // 2):
                    pk = rowsbf_v[b, r, pl.ds(v * 2 * LANES, 2 * LANES)]
                    lo, hi = plsc.unpack(pk,
                                         format=plsc.PackFormat.INTERLEAVED)
                    rowsf_v[r, pl.ds(v * 2 * LANES, LANES)] = lo * w
                    rowsf_v[r, pl.ds(v * 2 * LANES + LANES, LANES)] = hi * w
            return 0
        lax.fori_loop(0, K // LANES, body, 0)

    nbt = nchunk // GCH

    def _batch(bt, _):
        pltpu.sync_copy(src_hbm.at[tid, pl.ds(bt * GCH * K, GCH * K)], src_v)
        pltpu.sync_copy(dst_hbm.at[tid, pl.ds(bt * GCH, GCH)], dst_v)
        _gissue(0, 0)

        def _pair(p, _):
            l0 = p * 2
            _gwait(0)
            _ex_compute(l0, 0)
            _gissue(l0 + 1, 1)

            @pl.when(p > 0)
            def _():
                _sdrain(1)
            _scale(0)
            _sissue(l0, 0)
            _gwait(1)
            _ex_compute(l0 + 1, 1)
            _gissue(l0 + 2, 0)
            _sdrain(0)
            _scale(1)
            _sissue(l0 + 1, 1)
            return 0

        lax.fori_loop(0, GCH // 2, _pair, 0)

        _gwait(0)
        _ex_compute(GCH - 1, 0)
        _sdrain(1)
        _scale(0)
        _sissue(GCH - 1, 0)
        _sdrain(0)
        return 0

    lax.fori_loop(0, nbt, _batch, 0)

    plsc.subcore_barrier()

    pltpu.sync_copy(outacc.at[pl.ds(s * span, span)],
                    outp_hbm.at[c, pl.ds(s * span, span)])
    pltpu.sync_copy(denacc.at[pl.ds(s * span, span)],
                    denp_hbm.at[c, pl.ds(s * span, span)])


def _make_sc_edge(n, e_cnt, d):
    ept = e_cnt // NW
    nchunk = ept // K
    npad = ((n + (NS * K) - 1) // (NS * K)) * (NS * K)
    mesh = plsc.VectorSubcoreMesh(core_axis_name="c", subcore_axis_name="s",
                                  num_cores=NC, num_subcores=NS)
    body = functools.partial(_sc_edge_body, npad, nchunk, d)
    return pl.kernel(
        body,
        out_type=[
            jax.ShapeDtypeStruct((NC, npad, d), jnp.float32),
            jax.ShapeDtypeStruct((NC, npad), jnp.float32),
        ],
        mesh=mesh,
        compiler_params=pltpu.CompilerParams(needs_layout_passes=False,
                                             use_tc_tiling_on_sc=False),
        scratch_types=[
            pltpu.VMEM((GCH * K,), jnp.int32),
            pltpu.VMEM((GCH, K), jnp.int32),
            pltpu.VMEM((n,), jnp.float32),
            pltpu.VMEM((n,), jnp.float32),
            pltpu.VMEM((2, K), jnp.float32),
            pltpu.VMEM((2, K, d), jnp.bfloat16),
            pltpu.VMEM((K, d), jnp.float32),
            pltpu.VMEM_SHARED((npad, d), jnp.float32),
            pltpu.VMEM_SHARED((npad,), jnp.float32),
            pltpu.SemaphoreType.DMA,
            pltpu.SemaphoreType.DMA,
            pltpu.SemaphoreType.DMA,
        ],
    )


def _tc1_body(x_ref, w_ref, asr_ref, adr_ref,
              h_ref, as_ref, ad_ref, se_ref):
    h = jnp.dot(x_ref[...], w_ref[...], preferred_element_type=jnp.float32)
    h_ref[...] = h
    a_s = jnp.sum(h * asr_ref[...], axis=1, keepdims=True)
    a_d = jnp.sum(h * adr_ref[...], axis=1, keepdims=True)
    as_ref[...] = a_s
    ad_ref[...] = a_d
    z = a_s + a_d
    se_ref[...] = jnp.exp(jnp.where(z > 0, z, NEG_SLOPE * z))


def _tc2_body(o0_ref, o1_ref, h1_ref, se_ref, d0_ref, d1_ref, b_ref,
              w_ref, asr_ref, adr_ref,
              h_ref, as_ref, ad_ref, se2_ref):
    se = se_ref[...]
    agg = o0_ref[...] + o1_ref[...] + se * h1_ref[...]
    den = d0_ref[...] + d1_ref[...] + se + 1e-16
    x2 = jnp.maximum(agg / den + b_ref[...], 0.0)
    h = jnp.dot(x2, w_ref[...], preferred_element_type=jnp.float32)
    h_ref[...] = h
    a_s = jnp.sum(h * asr_ref[...], axis=1, keepdims=True)
    a_d = jnp.sum(h * adr_ref[...], axis=1, keepdims=True)
    as_ref[...] = a_s
    ad_ref[...] = a_d
    z = a_s + a_d
    se2_ref[...] = jnp.exp(jnp.where(z > 0, z, NEG_SLOPE * z))


def _tc3_body(o0_ref, o1_ref, h2_ref, se_ref, d0_ref, d1_ref, b_ref,
              out_ref):
    se = se_ref[...]
    agg = o0_ref[...] + o1_ref[...] + se * h2_ref[...]
    den = d0_ref[...] + d1_ref[...] + se + 1e-16
    out_ref[...] = agg / den + b_ref[...]


def _row_spec(b, d):
    return pl.BlockSpec((b, d), lambda i: (i, 0))


def _full_spec(r, c):
    return pl.BlockSpec((r, c), lambda i: (0, 0))


def kernel(X, E, W1, a_src1, a_dst1, b1, W2, a_src2, a_dst2, b2):
    n, d_in = X.shape
    d = W1.shape[1]
    e_cnt = E.shape[1]
    ept = e_cnt // NW
    nchunk = ept // K
    npad = ((n + (NS * K) - 1) // (NS * K)) * (NS * K)
    B = 1000
    grid = (n // B,)

    E = E.astype(jnp.int32)
    src32 = E[0].reshape(NW, ept)
    dst3 = E[1].reshape(NW, nchunk, K)

    perm = jnp.array([q * 32 + (j // 2 if j % 2 == 0 else 16 + j // 2)
                      for q in range(d // 32) for j in range(32)],
                     dtype=jnp.int32)

    tc1 = pl.pallas_call(
        _tc1_body,
        grid=grid,
        in_specs=[_row_spec(B, d_in), _full_spec(d_in, d),
                  _full_spec(1, d), _full_spec(1, d)],
        out_specs=[_row_spec(B, d), _row_spec(B, 1), _row_spec(B, 1),
                   _row_spec(B, 1)],
        out_shape=[jax.ShapeDtypeStruct((n, d), jnp.float32),
                   jax.ShapeDtypeStruct((n, 1), jnp.float32),
                   jax.ShapeDtypeStruct((n, 1), jnp.float32),
                   jax.ShapeDtypeStruct((n, 1), jnp.float32)],
    )
    h1, as1, ad1, se1 = tc1(X, W1, a_src1.reshape(1, d), a_dst1.reshape(1, d))

    sc_edge = _make_sc_edge(n, e_cnt, d)
    hbf1 = h1[:, perm].astype(jnp.bfloat16)
    outp1, denp1 = sc_edge(src32, dst3, as1.reshape(n), ad1.reshape(n), hbf1)

    tc2 = pl.pallas_call(
        _tc2_body,
        grid=grid,
        in_specs=[_row_spec(B, d), _row_spec(B, d), _row_spec(B, d),
                  _row_spec(B, 1), _row_spec(B, 1), _row_spec(B, 1),
                  _full_spec(1, d), _full_spec(d, d),
                  _full_spec(1, d), _full_spec(1, d)],
        out_specs=[_row_spec(B, d), _row_spec(B, 1), _row_spec(B, 1),
                   _row_spec(B, 1)],
        out_shape=[jax.ShapeDtypeStruct((n, d), jnp.float32),
                   jax.ShapeDtypeStruct((n, 1), jnp.float32),
                   jax.ShapeDtypeStruct((n, 1), jnp.float32),
                   jax.ShapeDtypeStruct((n, 1), jnp.float32)],
    )
    h2, as2, ad2, se2 = tc2(
        outp1[0], outp1[1], h1, se1,
        denp1[0].reshape(npad, 1), denp1[1].reshape(npad, 1),
        b1.reshape(1, d), W2,
        a_src2.reshape(1, d), a_dst2.reshape(1, d))

    hbf2 = h2[:, perm].astype(jnp.bfloat16)
    outp2, denp2 = sc_edge(src32, dst3, as2.reshape(n), ad2.reshape(n), hbf2)

    tc3 = pl.pallas_call(
        _tc3_body,
        grid=grid,
        in_specs=[_row_spec(B, d), _row_spec(B, d), _row_spec(B, d),
                  _row_spec(B, 1), _row_spec(B, 1), _row_spec(B, 1),
                  _full_spec(1, d)],
        out_specs=[_row_spec(B, d)],
        out_shape=[jax.ShapeDtypeStruct((n, d), jnp.float32)],
    )
    (out,) = tc3(
        outp2[0], outp2[1], h2, se2,
        denp2[0].reshape(npad, 1), denp2[1].reshape(npad, 1),
        b2.reshape(1, d))
    return out


# speedup vs baseline: 1.6348x; 1.6348x over previous
import functools

import jax
import jax.numpy as jnp
from jax import lax
from jax.experimental import pallas as pl
from jax.experimental.pallas import tpu as pltpu
from jax.experimental.pallas import tpu_sc as plsc

NC = 2
NS = 16
NW = NC * NS
LANES = 16

K = 80
GCH = 25
NEG_SLOPE = 0.2


def _sc_edge_body(npad, nchunk, d,
                  src_hbm, dst_hbm, as_hbm, ad_hbm, h_hbm,
                  outp_hbm, denp_hbm,
                  src_v, dst_v, as_v, ad_v, ex_v, rows_v,
                  outacc, denacc, gsem0, gsem1, ssem0, ssem1):
    c = lax.axis_index("c")
    s = lax.axis_index("s")
    tid = c * NS + s
    span = npad // NS
    nvec = 128 // LANES
    gsem = (gsem0, gsem1)
    ssem = (ssem0, ssem1)

    zv = jnp.zeros((LANES,), jnp.float32)

    def _zrows(r, _):
        for v in range(nvec):
            rows_v[0, r, pl.ds(v * LANES, LANES)] = zv
        return 0
    lax.fori_loop(0, K, _zrows, 0)

    def _zex(j, _):
        ex_v[0, pl.ds(j * LANES, LANES)] = zv
        return 0
    lax.fori_loop(0, K // LANES, _zex, 0)

    def _zacc(i, _):
        pltpu.sync_copy(rows_v.at[0], outacc.at[pl.ds(s * span + i * K, K)])
        return 0
    lax.fori_loop(0, span // K, _zacc, 0)

    def _zden(i, _):
        pltpu.sync_copy(ex_v.at[0], denacc.at[pl.ds(s * span + i * K, K)])
        return 0
    lax.fori_loop(0, span // K, _zden, 0)

    pltpu.sync_copy(as_hbm, as_v)
    pltpu.sync_copy(ad_hbm, ad_v)

    plsc.subcore_barrier()


    def _gissue(l, b):
        pltpu.async_copy(h_hbm.at[src_v.at[pl.ds(l * K, K)]],
                         rows_v.at[b], gsem[b])

    def _gwait(b):
        pltpu.make_async_copy(h_hbm.at[src_v.at[pl.ds(0, K)]],
                              rows_v.at[b], gsem[b]).wait()

    def _sissue(l, b):
        pltpu.async_copy(rows_v.at[b], outacc.at[dst_v.at[l]], ssem[b],
                         add=True)
        pltpu.async_copy(ex_v.at[b], denacc.at[dst_v.at[l]], ssem[b],
                         add=True)

    def _sdrain(b):
        pltpu.make_async_copy(rows_v.at[b], outacc.at[dst_v.at[0]],
                              ssem[b]).wait()
        pltpu.make_async_copy(ex_v.at[b], denacc.at[dst_v.at[0]],
                              ssem[b]).wait()

    def _ex_compute(l, b):
        for j in range(K // LANES):
            s16 = src_v[pl.ds(l * K + j * LANES, LANES)]
            d16 = dst_v[l, pl.ds(j * LANES, LANES)]
            av = plsc.load_gather(as_v, [s16])
            dv = plsc.load_gather(ad_v, [d16])
            z = av + dv
            e = jnp.where(z > 0, z, NEG_SLOPE * z)
            ex_v[b, pl.ds(j * LANES, LANES)] = jnp.exp(e)

    def _scale(b):
        def body(q, _):
            exg = ex_v[b, pl.ds(q * LANES, LANES)]
            for t in range(LANES):
                w = exg[t]
                r = q * LANES + t
                for v in range(nvec):
                    sl = pl.ds(v * LANES, LANES)
                    rows_v[b, r, sl] = rows_v[b, r, sl] * w
            return 0
        lax.fori_loop(0, K // LANES, body, 0)

    nbt = nchunk // GCH

    def _batch(bt, _):
        pltpu.sync_copy(src_hbm.at[tid, pl.ds(bt * GCH * K, GCH * K)], src_v)
        pltpu.sync_copy(dst_hbm.at[tid, pl.ds(bt * GCH, GCH)], dst_v)
        _gissue(0, 0)

        def _pair(p, _):
            l0 = p * 2
            _gwait(0)
            _ex_compute(l0, 0)

            @pl.when(p > 0)
            def _():
                _sdrain(1)
            _gissue(l0 + 1, 1)
            _scale(0)
            _sissue(l0, 0)
            _gwait(1)
            _ex_compute(l0 + 1, 1)
            _sdrain(0)
            _gissue(l0 + 2, 0)
            _scale(1)
            _sissue(l0 + 1, 1)
            return 0

        lax.fori_loop(0, GCH // 2, _pair, 0)

        _gwait(0)
        _ex_compute(GCH - 1, 0)
        _sdrain(1)
        _scale(0)
        _sissue(GCH - 1, 0)
        _sdrain(0)
        return 0

    lax.fori_loop(0, nbt, _batch, 0)

    plsc.subcore_barrier()

    pltpu.sync_copy(outacc.at[pl.ds(s * span, span)],
                    outp_hbm.at[c, pl.ds(s * span, span)])
    pltpu.sync_copy(denacc.at[pl.ds(s * span, span)],
                    denp_hbm.at[c, pl.ds(s * span, span)])


def _make_sc_edge(n, e_cnt, d):
    ept = e_cnt // NW
    nchunk = ept // K
    npad = ((n + (NS * K) - 1) // (NS * K)) * (NS * K)
    mesh = plsc.VectorSubcoreMesh(core_axis_name="c", subcore_axis_name="s",
                                  num_cores=NC, num_subcores=NS)
    body = functools.partial(_sc_edge_body, npad, nchunk, d)
    return pl.kernel(
        body,
        out_type=[
            jax.ShapeDtypeStruct((NC, npad, d), jnp.float32),
            jax.ShapeDtypeStruct((NC, npad), jnp.float32),
        ],
        mesh=mesh,
        compiler_params=pltpu.CompilerParams(needs_layout_passes=False,
                                             use_tc_tiling_on_sc=False),
        scratch_types=[
            pltpu.VMEM((GCH * K,), jnp.int32),
            pltpu.VMEM((GCH, K), jnp.int32),
            pltpu.VMEM((n,), jnp.float32),
            pltpu.VMEM((n,), jnp.float32),
            pltpu.VMEM((2, K), jnp.float32),
            pltpu.VMEM((2, K, d), jnp.float32),
            pltpu.VMEM_SHARED((npad, d), jnp.float32),
            pltpu.VMEM_SHARED((npad,), jnp.float32),
            pltpu.SemaphoreType.DMA,
            pltpu.SemaphoreType.DMA,
            pltpu.SemaphoreType.DMA,
            pltpu.SemaphoreType.DMA,
        ],
    )


def _tc1_body(x_ref, w_ref, asr_ref, adr_ref,
              h_ref, as_ref, ad_ref, se_ref):
    h = jnp.dot(x_ref[...], w_ref[...], preferred_element_type=jnp.float32)
    h_ref[...] = h
    a_s = jnp.sum(h * asr_ref[...], axis=1, keepdims=True)
    a_d = jnp.sum(h * adr_ref[...], axis=1, keepdims=True)
    as_ref[...] = a_s
    ad_ref[...] = a_d
    z = a_s + a_d
    se_ref[...] = jnp.exp(jnp.where(z > 0, z, NEG_SLOPE * z))


def _tc2_body(o0_ref, o1_ref, h1_ref, se_ref, d0_ref, d1_ref, b_ref,
              w_ref, asr_ref, adr_ref,
              h_ref, as_ref, ad_ref, se2_ref):
    se = se_ref[...]
    agg = o0_ref[...] + o1_ref[...] + se * h1_ref[...]
    den = d0_ref[...] + d1_ref[...] + se + 1e-16
    x2 = jnp.maximum(agg / den + b_ref[...], 0.0)
    h = jnp.dot(x2, w_ref[...], preferred_element_type=jnp.float32)
    h_ref[...] = h
    a_s = jnp.sum(h * asr_ref[...], axis=1, keepdims=True)
    a_d = jnp.sum(h * adr_ref[...], axis=1, keepdims=True)
    as_ref[...] = a_s
    ad_ref[...] = a_d
    z = a_s + a_d
    se2_ref[...] = jnp.exp(jnp.where(z > 0, z, NEG_SLOPE * z))


def _tc3_body(o0_ref, o1_ref, h2_ref, se_ref, d0_ref, d1_ref, b_ref,
              out_ref):
    se = se_ref[...]
    agg = o0_ref[...] + o1_ref[...] + se * h2_ref[...]
    den = d0_ref[...] + d1_ref[...] + se + 1e-16
    out_ref[...] = agg / den + b_ref[...]


def _row_spec(b, d):
    return pl.BlockSpec((b, d), lambda i: (i, 0))


def _full_spec(r, c):
    return pl.BlockSpec((r, c), lambda i: (0, 0))


def kernel(X, E, W1, a_src1, a_dst1, b1, W2, a_src2, a_dst2, b2):
    n, d_in = X.shape
    d = W1.shape[1]
    e_cnt = E.shape[1]
    ept = e_cnt // NW
    nchunk = ept // K
    npad = ((n + (NS * K) - 1) // (NS * K)) * (NS * K)
    B = 1000
    grid = (n // B,)

    E = E.astype(jnp.int32)
    src32 = E[0].reshape(NW, ept)
    dst3 = E[1].reshape(NW, nchunk, K)

    tc1 = pl.pallas_call(
        _tc1_body,
        grid=grid,
        in_specs=[_row_spec(B, d_in), _full_spec(d_in, d),
                  _full_spec(1, d), _full_spec(1, d)],
        out_specs=[_row_spec(B, d), _row_spec(B, 1), _row_spec(B, 1),
                   _row_spec(B, 1)],
        out_shape=[jax.ShapeDtypeStruct((n, d), jnp.float32),
                   jax.ShapeDtypeStruct((n, 1), jnp.float32),
                   jax.ShapeDtypeStruct((n, 1), jnp.float32),
                   jax.ShapeDtypeStruct((n, 1), jnp.float32)],
    )
    h1, as1, ad1, se1 = tc1(X, W1, a_src1.reshape(1, d), a_dst1.reshape(1, d))

    sc_edge = _make_sc_edge(n, e_cnt, d)
    outp1, denp1 = sc_edge(src32, dst3, as1.reshape(n), ad1.reshape(n), h1)

    tc2 = pl.pallas_call(
        _tc2_body,
        grid=grid,
        in_specs=[_row_spec(B, d), _row_spec(B, d), _row_spec(B, d),
                  _row_spec(B, 1), _row_spec(B, 1), _row_spec(B, 1),
                  _full_spec(1, d), _full_spec(d, d),
                  _full_spec(1, d), _full_spec(1, d)],
        out_specs=[_row_spec(B, d), _row_spec(B, 1), _row_spec(B, 1),
                   _row_spec(B, 1)],
        out_shape=[jax.ShapeDtypeStruct((n, d), jnp.float32),
                   jax.ShapeDtypeStruct((n, 1), jnp.float32),
                   jax.ShapeDtypeStruct((n, 1), jnp.float32),
                   jax.ShapeDtypeStruct((n, 1), jnp.float32)],
    )
    h2, as2, ad2, se2 = tc2(
        outp1[0], outp1[1], h1, se1,
        denp1[0].reshape(npad, 1), denp1[1].reshape(npad, 1),
        b1.reshape(1, d), W2,
        a_src2.reshape(1, d), a_dst2.reshape(1, d))

    outp2, denp2 = sc_edge(src32, dst3, as2.reshape(n), ad2.reshape(n), h2)

    tc3 = pl.pallas_call(
        _tc3_body,
        grid=grid,
        in_specs=[_row_spec(B, d), _row_spec(B, d), _row_spec(B, d),
                  _row_spec(B, 1), _row_spec(B, 1), _row_spec(B, 1),
                  _full_spec(1, d)],
        out_specs=[_row_spec(B, d)],
        out_shape=[jax.ShapeDtypeStruct((n, d), jnp.float32)],
    )
    (out,) = tc3(
        outp2[0], outp2[1], h2, se2,
        denp2[0].reshape(npad, 1), denp2[1].reshape(npad, 1),
        b2.reshape(1, d))
    return out
